# Initial kernel scaffold; baseline (speedup 1.0000x reference)
import jax
import jax.numpy as jnp
from jax.experimental import pallas as pl


def kernel(x, up_index, down_index, boundary_index, W_up, W_down, W_b, bias):
    raise NotImplementedError("write your pallas kernel here")


# speedup vs baseline: 2.5646x; 2.5646x over previous
import functools

import jax
import jax.numpy as jnp
from jax import lax
from jax.experimental import pallas as pl
from jax.experimental.pallas import tpu as pltpu
from jax.experimental.pallas import tpu_sc as plsc

N = 10000
D = 128
NC = 2
NS = 16
NW = NC * NS
CH = 128
EDGE_ALIGN = NW * CH * 2
N_PAD = 10112
ROWS_PER_TILE = N_PAD // NS
KSTG = 40


def _matmul_body(x_ref, wu_ref, wd_ref, wb_ref, yu_ref, yd_ref, yb_ref):
    xb = x_ref[...]
    yu_ref[...] = jnp.dot(xb, wu_ref[...], preferred_element_type=jnp.float32,
                          precision=lax.Precision.HIGHEST)
    yd_ref[...] = jnp.dot(xb, wd_ref[...], preferred_element_type=jnp.float32,
                          precision=lax.Precision.HIGHEST)
    yb_ref[...] = jnp.dot(xb, wb_ref[...], preferred_element_type=jnp.float32,
                          precision=lax.Precision.HIGHEST)


def _transform(x, W_up, W_down, W_b):
    blk = 1000
    grid = N // blk
    w_spec = pl.BlockSpec((D, D), lambda i: (0, 0))
    row_spec = pl.BlockSpec((blk, D), lambda i: (i, 0))
    return pl.pallas_call(
        _matmul_body,
        grid=(grid,),
        in_specs=[row_spec, w_spec, w_spec, w_spec],
        out_specs=[row_spec, row_spec, row_spec],
        out_shape=[jax.ShapeDtypeStruct((N, D), jnp.float32)] * 3,
    )(x, W_up, W_down, W_b)


def _sc_scatter_body(yu, yd, yb, su, du, sd, dd, sb, db, zeros,
                     p0, p1, acc, idx_s, idx_d, buf0, buf1, sem0, sem1):
    c = lax.axis_index("c")
    s = lax.axis_index("s")
    wid = s * NC + c

    pltpu.sync_copy(zeros, acc.at[pl.ds(s * ROWS_PER_TILE, ROWS_PER_TILE)])
    plsc.subcore_barrier()

    def run_stage(y, src_hbm, dst_hbm, base, k_rows):
        pltpu.sync_copy(src_hbm.at[pl.ds(base, k_rows)], idx_s.at[pl.ds(0, k_rows)])
        pltpu.sync_copy(dst_hbm.at[pl.ds(base, k_rows)], idx_d.at[pl.ds(0, k_rows)])

        def g_start(j, buf, sem):
            pltpu.async_copy(y.at[idx_s.at[j]], buf, sem)

        def g_wait(buf, sem):
            pltpu.make_async_copy(y.at[idx_s.at[0]], buf, sem).wait()

        g_start(0, buf0, sem0)
        g_start(1, buf1, sem1)

        def body(i, _):
            j0 = 2 * i
            g_wait(buf0, sem0)
            pltpu.sync_copy(buf0, acc.at[idx_d.at[j0]], add=True)

            @pl.when(j0 + 2 < k_rows)
            def _():
                g_start(j0 + 2, buf0, sem0)

            g_wait(buf1, sem1)
            pltpu.sync_copy(buf1, acc.at[idx_d.at[j0 + 1]], add=True)

            @pl.when(j0 + 3 < k_rows)
            def _():
                g_start(j0 + 3, buf1, sem1)

            return _

        lax.fori_loop(0, k_rows // 2, body, None)

    def run_table(y, src_hbm, dst_hbm):
        k_total = src_hbm.shape[0] // NW
        for st in range(0, k_total, KSTG):
            k = min(KSTG, k_total - st)
            run_stage(y, src_hbm, dst_hbm, wid * k_total + st, k)

    run_table(yu, su, du)
    run_table(yd, sd, dd)
    run_table(yb, sb, db)

    plsc.subcore_barrier()
    rows = pl.ds(s * ROWS_PER_TILE, ROWS_PER_TILE)

    @pl.when(c == 0)
    def _():
        pltpu.sync_copy(acc.at[rows], p0.at[rows])

    @pl.when(c == 1)
    def _():
        pltpu.sync_copy(acc.at[rows], p1.at[rows])


def _sc_scatter(yu, yd, yb, su, du, sd, dd, sb, db, zeros):
    kmax = KSTG
    mesh = plsc.VectorSubcoreMesh(core_axis_name="c", subcore_axis_name="s")
    f = pl.kernel(
        _sc_scatter_body,
        out_type=(jax.ShapeDtypeStruct((N_PAD, D), jnp.float32),
                  jax.ShapeDtypeStruct((N_PAD, D), jnp.float32)),
        mesh=mesh,
        scratch_types=[
            pltpu.VMEM_SHARED((N_PAD, D), jnp.float32),
            pltpu.VMEM((kmax, CH), jnp.int32),
            pltpu.VMEM((kmax, CH), jnp.int32),
            pltpu.VMEM((CH, D), jnp.float32),
            pltpu.VMEM((CH, D), jnp.float32),
            pltpu.SemaphoreType.DMA,
            pltpu.SemaphoreType.DMA,
        ],
    )
    return f(yu, yd, yb, su, du, sd, dd, sb, db, zeros)


def _combine_body(p0_ref, p1_ref, b_ref, o_ref):
    o_ref[...] = p0_ref[...] + p1_ref[...] + b_ref[...]


def _combine(p0, p1, bias):
    blk = 1000
    row_spec = pl.BlockSpec((blk, D), lambda i: (i, 0))
    return pl.pallas_call(
        _combine_body,
        grid=(N // blk,),
        in_specs=[row_spec, row_spec, pl.BlockSpec((1, D), lambda i: (0, 0))],
        out_specs=row_spec,
        out_shape=jax.ShapeDtypeStruct((N, D), jnp.float32),
    )(p0, p1, bias)


def _pad_edges(row, pad_val):
    e = row.shape[0]
    e_pad = -(-e // EDGE_ALIGN) * EDGE_ALIGN
    pad = jnp.full((e_pad - e,), pad_val, jnp.int32)
    return jnp.concatenate([row.astype(jnp.int32), pad]).reshape(-1, CH)


def kernel(x, up_index, down_index, boundary_index, W_up, W_down, W_b, bias):
    su = _pad_edges(up_index[0], 0)
    du = _pad_edges(up_index[1], N)
    sd = _pad_edges(down_index[0], 0)
    dd = _pad_edges(down_index[1], N)
    sb = _pad_edges(boundary_index[0], 0)
    db = _pad_edges(boundary_index[1], N)
    zeros = jnp.zeros((ROWS_PER_TILE, D), jnp.float32)

    yu, yd, yb = _transform(x, W_up, W_down, W_b)
    p0, p1 = _sc_scatter(yu, yd, yb, su, du, sd, dd, sb, db, zeros)
    return _combine(p0, p1, bias.reshape(1, D))
